# baseline (device time: 23894 ns/iter reference)
import jax
import jax.numpy as jnp
from jax import lax
from jax.experimental import pallas as pl
from jax.experimental.pallas import tpu as pltpu

N_DEV = 4
_GELU_C = 0.7978845608028654
_DESTS = (2, 1, 3, 0)
_DRAIN = ((1, 0), (1, 1), (3, 0), (3, 1), (2, 0), (2, 1))


def _gelu(y):
    return 0.5 * y * (1.0 + jnp.tanh(_GELU_C * (y + 0.044715 * y * y * y)))


def kernel(x, w_mat):
    m_per, k = x.shape
    _, n = w_mat.shape
    n_per = n // N_DEV
    m_half = m_per // 2

    def body(
        x_hbm, w_hbm, out_hbm,
        x_vmem, wbuf, snd, rcv, stage,
        x_sem, w_sems, out_sems, send_sems, recv_sems,
    ):
        my_pos = lax.axis_index("i")

        def wcopy(d, slot):
            tgt = (my_pos + d) % N_DEV
            return pltpu.make_async_copy(
                w_hbm.at[:, pl.ds(tgt * n_per, n_per)],
                wbuf.at[slot],
                w_sems.at[slot],
            )

        xcopy = pltpu.make_async_copy(x_hbm, x_vmem, x_sem)
        xcopy.start()
        wcopy(_DESTS[0], 0).start()

        barrier_sem = pltpu.get_barrier_semaphore()
        for d in range(1, N_DEV):
            pl.semaphore_signal(
                barrier_sem,
                inc=1,
                device_id=((my_pos + d) % N_DEV,),
                device_id_type=pl.DeviceIdType.MESH,
            )
        pl.semaphore_wait(barrier_sem, N_DEV - 1)
        xcopy.wait()

        out_dma = {0: None, 1: None}
        out_uses = [0]

        def stage_out(piece_f32, row_q):
            slot = out_uses[0] % 2
            out_uses[0] += 1
            if out_dma[slot] is not None:
                out_dma[slot].wait()
            stage[slot] = piece_f32
            dma = pltpu.make_async_copy(
                stage.at[slot],
                out_hbm.at[pl.ds(row_q * m_half, m_half), :],
                out_sems.at[slot],
            )
            dma.start()
            out_dma[slot] = dma

        rdmas = {}
        for s, d in enumerate(_DESTS):
            if s + 1 < N_DEV:
                wcopy(_DESTS[s + 1], (s + 1) % 2).start()
            wcopy(d, s % 2).wait()
            for r in (0, 1):
                y_half = _gelu(
                    jnp.dot(
                        x_vmem[r * m_half : (r + 1) * m_half, :],
                        wbuf[s % 2],
                        preferred_element_type=jnp.float32,
                    )
                )
                if d == 0:
                    stage_out(y_half, 2 * my_pos + r)
                else:
                    idx = (d - 1) * 2 + r
                    snd[idx] = y_half.astype(jnp.bfloat16)
                    rdma = pltpu.make_async_remote_copy(
                        src_ref=snd.at[idx],
                        dst_ref=rcv.at[idx],
                        send_sem=send_sems.at[idx],
                        recv_sem=recv_sems.at[idx],
                        device_id=((my_pos + d) % N_DEV,),
                        device_id_type=pl.DeviceIdType.MESH,
                    )
                    rdma.start()
                    rdmas[(d, r)] = rdma

        for d, r in _DRAIN:
            rdmas[(d, r)].wait()
            src_pos = (my_pos - d) % N_DEV
            stage_out(
                rcv[(d - 1) * 2 + r].astype(jnp.float32), 2 * src_pos + r
            )
        for slot in (0, 1):
            if out_dma[slot] is not None:
                out_dma[slot].wait()

    return pl.pallas_call(
        body,
        out_shape=jax.ShapeDtypeStruct((N_DEV * m_per, n_per), jnp.float32),
        in_specs=[
            pl.BlockSpec(memory_space=pltpu.MemorySpace.HBM),
            pl.BlockSpec(memory_space=pltpu.MemorySpace.HBM),
        ],
        out_specs=pl.BlockSpec(memory_space=pltpu.MemorySpace.HBM),
        scratch_shapes=[
            pltpu.VMEM((m_per, k), jnp.float32),
            pltpu.VMEM((2, k, n_per), jnp.float32),
            pltpu.VMEM((6, m_half, n_per), jnp.bfloat16),
            pltpu.VMEM((6, m_half, n_per), jnp.bfloat16),
            pltpu.VMEM((2, m_half, n_per), jnp.float32),
            pltpu.SemaphoreType.DMA,
            pltpu.SemaphoreType.DMA((2,)),
            pltpu.SemaphoreType.DMA((2,)),
            pltpu.SemaphoreType.DMA((6,)),
            pltpu.SemaphoreType.DMA((6,)),
        ],
        compiler_params=pltpu.CompilerParams(collective_id=0),
    )(x, w_mat)
